# baseline (speedup 1.0000x reference)
import functools

import jax
import jax.numpy as jnp
from jax import lax
from jax.experimental import pallas as pl
from jax.experimental.pallas import tpu as pltpu

N = 16384
SUM = 32
DIM = 128
BN = 128
NBLK = N // BN
K = 8


def _score_body(q_ref, s_ref, w_ref, idx_ref, scores_ref, vals_ref):
    i = pl.program_id(0)
    blk = s_ref[...]
    m = jnp.mean(blk, axis=1)

    q = q_ref[0]
    qm = jnp.mean(q, axis=0, keepdims=True)
    qn = qm / (jnp.sqrt(jnp.sum(qm * qm)) + 1e-6)
    q_len = jnp.sqrt(jnp.sum(qn * qn))

    dims = (((1,), (1,)), ((), ()))
    dot = lax.dot_general(qn, m, dims,
                          precision=lax.Precision.HIGHEST,
                          preferred_element_type=jnp.float32)
    ones = jnp.ones((1, DIM), jnp.float32)
    n2 = lax.dot_general(ones, m * m, dims,
                         precision=lax.Precision.HIGHEST,
                         preferred_element_type=jnp.float32)
    sn = jnp.sqrt(n2)
    f = 1.0 / (sn + 1e-6)
    num = dot * f
    den = jnp.maximum(q_len * (sn * f), 1e-8)
    scores_ref[pl.ds(i, 1), :] = num / den

    @pl.when(i == NBLK - 1)
    def _():
        S = scores_ref[...]
        ids = (lax.broadcasted_iota(jnp.int32, (NBLK, BN), 0) * BN
               + lax.broadcasted_iota(jnp.int32, (NBLK, BN), 1))
        for t in range(K):
            v = jnp.max(S)
            sel = jnp.min(jnp.where(S == v, ids, jnp.int32(2**30)))
            idx_ref[pl.ds(t, 1), :] = jnp.full((1, BN), sel, jnp.int32)
            vals_ref[pl.ds(t, 1), :] = jnp.full((1, BN), v, jnp.float32)
            S = jnp.where(ids == sel, jnp.float32(-1e30), S)
        V = vals_ref[...]
        E = jnp.exp(V - jnp.max(V, axis=0, keepdims=True))
        w_ref[...] = E / jnp.sum(E, axis=0, keepdims=True)


def _combine_body(idx_ref, s_ref, w_ref, o_ref):
    @pl.when(pl.program_id(0) == 0)
    def _():
        o_ref[...] = jnp.zeros_like(o_ref)
    o_ref[...] += s_ref[0] * w_ref[0]


@jax.jit
def _run(query, summaries):
    w_mat, idx_mat = pl.pallas_call(
        _score_body,
        grid=(NBLK,),
        in_specs=[
            pl.BlockSpec((1, 64, DIM), lambda i: (0, 0, 0)),
            pl.BlockSpec((BN, SUM, DIM), lambda i: (i, 0, 0)),
        ],
        out_specs=[
            pl.BlockSpec((K, BN), lambda i: (0, 0)),
            pl.BlockSpec((K, BN), lambda i: (0, 0)),
        ],
        out_shape=[
            jax.ShapeDtypeStruct((K, BN), jnp.float32),
            jax.ShapeDtypeStruct((K, BN), jnp.int32),
        ],
        scratch_shapes=[
            pltpu.VMEM((NBLK, BN), jnp.float32),
            pltpu.VMEM((K, BN), jnp.float32),
        ],
    )(query, summaries)

    top_idx = idx_mat[:, 0]
    w3 = w_mat.reshape(K, 1, BN)

    out = pl.pallas_call(
        _combine_body,
        grid_spec=pltpu.PrefetchScalarGridSpec(
            num_scalar_prefetch=1,
            grid=(K,),
            in_specs=[
                pl.BlockSpec((1, SUM, DIM), lambda i, idx: (idx[i], 0, 0)),
                pl.BlockSpec((1, 1, BN), lambda i, idx: (i, 0, 0)),
            ],
            out_specs=pl.BlockSpec((SUM, DIM), lambda i, idx: (0, 0)),
        ),
        out_shape=jax.ShapeDtypeStruct((SUM, DIM), jnp.float32),
    )(top_idx, summaries, w3)
    return out


def kernel(query, summaries, k):
    del k
    return _run(query, summaries)


# speedup vs baseline: 1.4245x; 1.4245x over previous
import functools

import jax
import jax.numpy as jnp
from jax import lax
from jax.experimental import pallas as pl
from jax.experimental.pallas import tpu as pltpu

N = 16384
SUM = 32
DIM = 128
BN = 256
NBLK = N // BN
K = 8


def _score_body(q_ref, s_ref, w_ref, idx_ref, scores_ref, vals_ref,
                qn_ref, ql_ref):
    i = pl.program_id(0)

    @pl.when(i == 0)
    def _():
        q = q_ref[0]
        qm = jnp.sum(q, axis=0, keepdims=True) * (1.0 / 64.0)
        qn = qm / (jnp.sqrt(jnp.sum(qm * qm)) + 1e-6)
        qn_ref[...] = qn
        ql_ref[...] = jnp.full((1, BN), jnp.sqrt(jnp.sum(qn * qn)))

    blk = s_ref[...]
    ssum = jnp.sum(blk, axis=1)

    dims = (((1,), (1,)), ((), ()))
    dot = lax.dot_general(qn_ref[...], ssum, dims,
                          precision=lax.Precision.HIGHEST,
                          preferred_element_type=jnp.float32) * (1.0 / SUM)
    ones = jnp.ones((1, DIM), jnp.float32)
    n2 = lax.dot_general(ones, ssum * ssum, dims,
                         precision=lax.Precision.HIGHEST,
                         preferred_element_type=jnp.float32) * (1.0 / (SUM * SUM))
    sn = jnp.sqrt(n2)
    f = 1.0 / (sn + 1e-6)
    num = dot * f
    den = jnp.maximum(ql_ref[...] * (sn * f), 1e-8)
    scores_ref[pl.ds(i, 1), :] = num / den

    @pl.when(i == NBLK - 1)
    def _():
        S = scores_ref[...]
        ids = (lax.broadcasted_iota(jnp.int32, (NBLK, BN), 0) * BN
               + lax.broadcasted_iota(jnp.int32, (NBLK, BN), 1))
        for t in range(K):
            v = jnp.max(S)
            sel = jnp.min(jnp.where(S == v, ids, jnp.int32(2**30)))
            idx_ref[pl.ds(t, 1), :] = jnp.full((1, BN), sel, jnp.int32)
            vals_ref[pl.ds(t, 1), :] = jnp.full((1, BN), v, jnp.float32)
            S = jnp.where(ids == sel, jnp.float32(-1e30), S)
        V = vals_ref[...]
        E = jnp.exp(V - jnp.max(V, axis=0, keepdims=True))
        w_ref[...] = E / jnp.sum(E, axis=0, keepdims=True)


def _combine_body(idx_ref, s_ref, w_ref, o_ref):
    @pl.when(pl.program_id(0) == 0)
    def _():
        o_ref[...] = jnp.zeros_like(o_ref)
    o_ref[...] += s_ref[0] * w_ref[0][:, :DIM]


@jax.jit
def _run(query, summaries):
    w_mat, idx_mat = pl.pallas_call(
        _score_body,
        grid=(NBLK,),
        in_specs=[
            pl.BlockSpec((1, 64, DIM), lambda i: (0, 0, 0)),
            pl.BlockSpec((BN, SUM, DIM), lambda i: (i, 0, 0)),
        ],
        out_specs=[
            pl.BlockSpec((K, BN), lambda i: (0, 0)),
            pl.BlockSpec((K, BN), lambda i: (0, 0)),
        ],
        out_shape=[
            jax.ShapeDtypeStruct((K, BN), jnp.float32),
            jax.ShapeDtypeStruct((K, BN), jnp.int32),
        ],
        scratch_shapes=[
            pltpu.VMEM((NBLK, BN), jnp.float32),
            pltpu.VMEM((K, BN), jnp.float32),
            pltpu.VMEM((1, DIM), jnp.float32),
            pltpu.VMEM((1, BN), jnp.float32),
        ],
    )(query, summaries)

    top_idx = idx_mat[:, 0]
    w3 = w_mat.reshape(K, 1, BN)

    out = pl.pallas_call(
        _combine_body,
        grid_spec=pltpu.PrefetchScalarGridSpec(
            num_scalar_prefetch=1,
            grid=(K,),
            in_specs=[
                pl.BlockSpec((1, SUM, DIM), lambda i, idx: (idx[i], 0, 0)),
                pl.BlockSpec((1, 1, BN), lambda i, idx: (i, 0, 0)),
            ],
            out_specs=pl.BlockSpec((SUM, DIM), lambda i, idx: (0, 0)),
        ),
        out_shape=jax.ShapeDtypeStruct((SUM, DIM), jnp.float32),
    )(top_idx, summaries, w3)
    return out


def kernel(query, summaries, k):
    del k
    return _run(query, summaries)


# speedup vs baseline: 1.6737x; 1.1750x over previous
import functools

import jax
import jax.numpy as jnp
from jax import lax
from jax.experimental import pallas as pl
from jax.experimental.pallas import tpu as pltpu

N = 16384
SUM = 32
DIM = 128
BN = 256
NBLK = N // BN
K = 8


def _scores_of(ssum, qn, ql):
    dims = (((1,), (1,)), ((), ()))
    dot = lax.dot_general(qn, ssum, dims,
                          precision=lax.Precision.HIGHEST,
                          preferred_element_type=jnp.float32) * (1.0 / SUM)
    ones = jnp.ones((1, DIM), jnp.float32)
    n2 = lax.dot_general(ones, ssum * ssum, dims,
                         precision=lax.Precision.HIGHEST,
                         preferred_element_type=jnp.float32) * (1.0 / (SUM * SUM))
    sn = jnp.sqrt(n2)
    f = 1.0 / (sn + 1e-6)
    num = dot * f
    den = jnp.maximum(ql * (sn * f), 1e-8)
    return num / den


def _score_body(q_ref, sa_ref, sb_ref, w_ref, idx_ref, scores_ref, vals_ref,
                qn_ref, ql_ref):
    i = pl.program_id(0)

    @pl.when(i == 0)
    def _():
        q = q_ref[0]
        qm = jnp.sum(q, axis=0, keepdims=True) * (1.0 / 64.0)
        qn = qm / (jnp.sqrt(jnp.sum(qm * qm)) + 1e-6)
        qn_ref[...] = qn
        ql_ref[...] = jnp.full((1, BN), jnp.sqrt(jnp.sum(qn * qn)))

    qn = qn_ref[...]
    ql = ql_ref[...]
    ssum_a = jnp.sum(sa_ref[...], axis=1)
    ssum_b = jnp.sum(sb_ref[...], axis=1)
    scores_ref[pl.ds(i, 1), :] = _scores_of(ssum_a, qn, ql)
    scores_ref[pl.ds(i + NBLK // 2, 1), :] = _scores_of(ssum_b, qn, ql)

    @pl.when(i == NBLK // 2 - 1)
    def _():
        S = scores_ref[...]
        ids = (lax.broadcasted_iota(jnp.int32, (NBLK, BN), 0) * BN
               + lax.broadcasted_iota(jnp.int32, (NBLK, BN), 1))
        for t in range(K):
            v = jnp.max(S)
            sel = jnp.min(jnp.where(S == v, ids, jnp.int32(2**30)))
            idx_ref[pl.ds(t, 1), :] = jnp.full((1, BN), sel, jnp.int32)
            vals_ref[pl.ds(t, 1), :] = jnp.full((1, BN), v, jnp.float32)
            S = jnp.where(ids == sel, jnp.float32(-1e30), S)
        V = vals_ref[...]
        E = jnp.exp(V - jnp.max(V, axis=0, keepdims=True))
        w_ref[...] = E / jnp.sum(E, axis=0, keepdims=True)


def _combine_body(idx_ref, s_ref, w_ref, o_ref):
    @pl.when(pl.program_id(0) == 0)
    def _():
        o_ref[...] = jnp.zeros_like(o_ref)
    o_ref[...] += s_ref[0] * w_ref[0][:, :DIM]


@jax.jit
def _run(query, summaries):
    w_mat, idx_mat = pl.pallas_call(
        _score_body,
        grid=(NBLK // 2,),
        in_specs=[
            pl.BlockSpec((1, 64, DIM), lambda i: (0, 0, 0)),
            pl.BlockSpec((BN, SUM, DIM), lambda i: (i, 0, 0)),
            pl.BlockSpec((BN, SUM, DIM), lambda i: (i + NBLK // 2, 0, 0)),
        ],
        out_specs=[
            pl.BlockSpec((K, BN), lambda i: (0, 0)),
            pl.BlockSpec((K, BN), lambda i: (0, 0)),
        ],
        out_shape=[
            jax.ShapeDtypeStruct((K, BN), jnp.float32),
            jax.ShapeDtypeStruct((K, BN), jnp.int32),
        ],
        scratch_shapes=[
            pltpu.VMEM((NBLK, BN), jnp.float32),
            pltpu.VMEM((K, BN), jnp.float32),
            pltpu.VMEM((1, DIM), jnp.float32),
            pltpu.VMEM((1, BN), jnp.float32),
        ],
    )(query, summaries, summaries)

    top_idx = idx_mat[:, 0]
    w3 = w_mat.reshape(K, 1, BN)

    out = pl.pallas_call(
        _combine_body,
        grid_spec=pltpu.PrefetchScalarGridSpec(
            num_scalar_prefetch=1,
            grid=(K,),
            in_specs=[
                pl.BlockSpec((1, SUM, DIM), lambda i, idx: (idx[i], 0, 0)),
                pl.BlockSpec((1, 1, BN), lambda i, idx: (i, 0, 0)),
            ],
            out_specs=pl.BlockSpec((SUM, DIM), lambda i, idx: (0, 0)),
        ),
        out_shape=jax.ShapeDtypeStruct((SUM, DIM), jnp.float32),
    )(top_idx, summaries, w3)
    return out


def kernel(query, summaries, k):
    del k
    return _run(query, summaries)


# speedup vs baseline: 1.6965x; 1.0136x over previous
import functools

import jax
import jax.numpy as jnp
from jax import lax
from jax.experimental import pallas as pl
from jax.experimental.pallas import tpu as pltpu

N = 16384
SUM = 32
DIM = 128
BN = 256
NBLK = N // BN
NSTREAM = 4
K = 8


def _scores_of(ssum, qn, ql):
    dims = (((1,), (1,)), ((), ()))
    dot = lax.dot_general(qn, ssum, dims,
                          precision=lax.Precision.HIGHEST,
                          preferred_element_type=jnp.float32) * (1.0 / SUM)
    ones = jnp.ones((1, DIM), jnp.float32)
    n2 = lax.dot_general(ones, ssum * ssum, dims,
                         precision=lax.Precision.HIGHEST,
                         preferred_element_type=jnp.float32) * (1.0 / (SUM * SUM))
    sn = jnp.sqrt(n2)
    f = 1.0 / (sn + 1e-6)
    num = dot * f
    den = jnp.maximum(ql * (sn * f), 1e-8)
    return num / den


def _score_body(q_ref, *refs):
    (s_refs, (w_ref, idx_ref, scores_ref, vals_ref, qn_ref, ql_ref)) = (
        refs[:NSTREAM], refs[NSTREAM:])
    i = pl.program_id(0)
    nstep = NBLK // NSTREAM

    @pl.when(i == 0)
    def _():
        q = q_ref[0]
        qm = jnp.sum(q, axis=0, keepdims=True) * (1.0 / 64.0)
        qn = qm / (jnp.sqrt(jnp.sum(qm * qm)) + 1e-6)
        qn_ref[...] = qn
        ql_ref[...] = jnp.full((1, BN), jnp.sqrt(jnp.sum(qn * qn)))

    qn = qn_ref[...]
    ql = ql_ref[...]
    for s, s_ref in enumerate(s_refs):
        ssum = jnp.sum(s_ref[...], axis=1)
        scores_ref[pl.ds(i + s * nstep, 1), :] = _scores_of(ssum, qn, ql)

    @pl.when(i == NBLK // NSTREAM - 1)
    def _():
        S = scores_ref[...]
        ids = (lax.broadcasted_iota(jnp.int32, (NBLK, BN), 0) * BN
               + lax.broadcasted_iota(jnp.int32, (NBLK, BN), 1))
        for t in range(K):
            v = jnp.max(S)
            sel = jnp.min(jnp.where(S == v, ids, jnp.int32(2**30)))
            idx_ref[pl.ds(t, 1), :] = jnp.full((1, BN), sel, jnp.int32)
            vals_ref[pl.ds(t, 1), :] = jnp.full((1, BN), v, jnp.float32)
            S = jnp.where(ids == sel, jnp.float32(-1e30), S)
        V = vals_ref[...]
        E = jnp.exp(V - jnp.max(V, axis=0, keepdims=True))
        w_ref[...] = E / jnp.sum(E, axis=0, keepdims=True)


def _combine_body(idx_ref, s_ref, w_ref, o_ref):
    @pl.when(pl.program_id(0) == 0)
    def _():
        o_ref[...] = jnp.zeros_like(o_ref)
    o_ref[...] += s_ref[0] * w_ref[0][:, :DIM]


@jax.jit
def _run(query, summaries):
    w_mat, idx_mat = pl.pallas_call(
        _score_body,
        grid=(NBLK // NSTREAM,),
        in_specs=[pl.BlockSpec((1, 64, DIM), lambda i: (0, 0, 0))] + [
            pl.BlockSpec((BN, SUM, DIM),
                         functools.partial(
                             lambda s, i: (i + s * (NBLK // NSTREAM), 0, 0), s))
            for s in range(NSTREAM)
        ],
        out_specs=[
            pl.BlockSpec((K, BN), lambda i: (0, 0)),
            pl.BlockSpec((K, BN), lambda i: (0, 0)),
        ],
        out_shape=[
            jax.ShapeDtypeStruct((K, BN), jnp.float32),
            jax.ShapeDtypeStruct((K, BN), jnp.int32),
        ],
        scratch_shapes=[
            pltpu.VMEM((NBLK, BN), jnp.float32),
            pltpu.VMEM((K, BN), jnp.float32),
            pltpu.VMEM((1, DIM), jnp.float32),
            pltpu.VMEM((1, BN), jnp.float32),
        ],
    )(query, *([summaries] * NSTREAM))

    top_idx = idx_mat[:, 0]
    w3 = w_mat.reshape(K, 1, BN)

    out = pl.pallas_call(
        _combine_body,
        grid_spec=pltpu.PrefetchScalarGridSpec(
            num_scalar_prefetch=1,
            grid=(K,),
            in_specs=[
                pl.BlockSpec((1, SUM, DIM), lambda i, idx: (idx[i], 0, 0)),
                pl.BlockSpec((1, 1, BN), lambda i, idx: (i, 0, 0)),
            ],
            out_specs=pl.BlockSpec((SUM, DIM), lambda i, idx: (0, 0)),
        ),
        out_shape=jax.ShapeDtypeStruct((SUM, DIM), jnp.float32),
    )(top_idx, summaries, w3)
    return out


def kernel(query, summaries, k):
    del k
    return _run(query, summaries)
